# baseline (speedup 1.0000x reference)
import functools

import jax
import jax.numpy as jnp
from jax import lax
from jax.experimental import pallas as pl
from jax.experimental.pallas import tpu as pltpu
from jax.experimental.pallas import tpu_sc as plsc

B = 16384
S = 50
D = 64
NC = 2
NS = 16
NW = NC * NS
RPW = B // NW
GSZ = 128
NG = RPW // GSZ
EPS = 0.001
L = 16
BB = 256


def _gather_body(cid_hbm, mean_hbm, std_hbm, m_hbm, r_hbm,
                 idx0, idx1, idx2, idx3, mrows, srows, sem_g, sem_o):
    wid = lax.axis_index("s") * NC + lax.axis_index("c")
    base = wid * RPW
    idxs = [idx0, idx1, idx2, idx3]

    for g in range(NG):
        pltpu.sync_copy(cid_hbm.at[pl.ds(base + g * GSZ, GSZ)], idxs[g])
    gathers = []
    for g in range(NG):
        dst_m = mrows.at[pl.ds(g * GSZ, GSZ)]
        dst_s = srows.at[pl.ds(g * GSZ, GSZ)]
        gathers.append(pltpu.async_copy(mean_hbm.at[idxs[g]], dst_m, sem_g))
        gathers.append(pltpu.async_copy(std_hbm.at[idxs[g]], dst_s, sem_g))
    for cp in gathers:
        cp.wait()

    out_m = pltpu.async_copy(mrows, m_hbm.at[pl.ds(base, RPW)], sem_o)

    @plsc.parallel_loop(0, RPW)
    def rinv_body(i):
        for j in range(D // L):
            v = srows[i, pl.ds(j * L, L)]
            srows[i, pl.ds(j * L, L)] = 1.0 / (jnp.exp(v) + EPS)

    out_r = pltpu.async_copy(srows, r_hbm.at[pl.ds(base, RPW)], sem_o)
    out_m.wait()
    out_r.wait()


def _sc_gather(cid, initial_mean, initial_std):
    mesh = plsc.VectorSubcoreMesh(core_axis_name="c", subcore_axis_name="s")
    run = functools.partial(
        pl.kernel,
        out_type=(
            jax.ShapeDtypeStruct((B, D), jnp.float32),
            jax.ShapeDtypeStruct((B, D), jnp.float32),
        ),
        mesh=mesh,
        compiler_params=pltpu.CompilerParams(use_tc_tiling_on_sc=False),
        scratch_types=[
            pltpu.VMEM((GSZ,), jnp.int32),
            pltpu.VMEM((GSZ,), jnp.int32),
            pltpu.VMEM((GSZ,), jnp.int32),
            pltpu.VMEM((GSZ,), jnp.int32),
            pltpu.VMEM((RPW, D), jnp.float32),
            pltpu.VMEM((RPW, D), jnp.float32),
            pltpu.SemaphoreType.DMA,
            pltpu.SemaphoreType.DMA,
        ],
    )(_gather_body)
    return run(cid, initial_mean, initial_std)


def _norm_body(x_ref, m_ref, r_ref, o_ref):
    m = m_ref[...][:, None, :]
    r = r_ref[...][:, None, :]
    o_ref[...] = (x_ref[...] - m) * r


def _tc_normalize(x, m, r):
    return pl.pallas_call(
        _norm_body,
        out_shape=jax.ShapeDtypeStruct((B, S, D), jnp.float32),
        grid=(B // BB,),
        in_specs=[
            pl.BlockSpec((BB, S, D), lambda i: (i, 0, 0)),
            pl.BlockSpec((BB, D), lambda i: (i, 0)),
            pl.BlockSpec((BB, D), lambda i: (i, 0)),
        ],
        out_specs=pl.BlockSpec((BB, S, D), lambda i: (i, 0, 0)),
    )(x, m, r)


@jax.jit
def kernel(x, cluster_id, initial_mean, initial_std):
    cid = cluster_id.reshape(B)
    m, r = _sc_gather(cid, initial_mean, initial_std)
    return _tc_normalize(x, m, r)


# speedup vs baseline: 5.6281x; 5.6281x over previous
import functools

import jax
import jax.numpy as jnp
from jax import lax
from jax.experimental import pallas as pl
from jax.experimental.pallas import tpu as pltpu
from jax.experimental.pallas import tpu_sc as plsc

B = 16384
S = 50
D = 64
DP = 128
NC = 2
NS = 16
NW = NC * NS
RPW = B // NW
GSZ = 128
NG = RPW // GSZ
EPS = 0.001
L = 16
BB = 512


def _gather_body(cid_hbm, mean_hbm, std_hbm, m_hbm, r_hbm,
                 idx0, idx1, idx2, idx3, mrows, srows, sem_g, sem_o):
    wid = lax.axis_index("s") * NC + lax.axis_index("c")
    base = wid * RPW
    idxs = [idx0, idx1, idx2, idx3]

    for g in range(NG):
        pltpu.sync_copy(cid_hbm.at[pl.ds(base + g * GSZ, GSZ)], idxs[g])
    gathers = []
    for g in range(NG):
        dst_m = mrows.at[pl.ds(g * GSZ, GSZ)]
        dst_s = srows.at[pl.ds(g * GSZ, GSZ)]
        gathers.append(pltpu.async_copy(mean_hbm.at[idxs[g]], dst_m, sem_g))
        gathers.append(pltpu.async_copy(std_hbm.at[idxs[g]], dst_s, sem_g))
    for cp in gathers:
        cp.wait()

    out_m = pltpu.async_copy(
        mrows, m_hbm.at[pl.ds(base, RPW), pl.ds(0, D)], sem_o)

    @plsc.parallel_loop(0, RPW)
    def rinv_body(i):
        for j in range(D // L):
            v = srows[i, pl.ds(j * L, L)]
            srows[i, pl.ds(j * L, L)] = 1.0 / (jnp.exp(v) + EPS)

    out_r = pltpu.async_copy(
        srows, r_hbm.at[pl.ds(base, RPW), pl.ds(0, D)], sem_o)
    out_m.wait()
    out_r.wait()


def _sc_gather(cid, initial_mean, initial_std):
    mesh = plsc.VectorSubcoreMesh(core_axis_name="c", subcore_axis_name="s")
    run = functools.partial(
        pl.kernel,
        out_type=(
            jax.ShapeDtypeStruct((B, DP), jnp.float32),
            jax.ShapeDtypeStruct((B, DP), jnp.float32),
        ),
        mesh=mesh,
        compiler_params=pltpu.CompilerParams(use_tc_tiling_on_sc=False),
        scratch_types=[
            pltpu.VMEM((GSZ,), jnp.int32),
            pltpu.VMEM((GSZ,), jnp.int32),
            pltpu.VMEM((GSZ,), jnp.int32),
            pltpu.VMEM((GSZ,), jnp.int32),
            pltpu.VMEM((RPW, D), jnp.float32),
            pltpu.VMEM((RPW, D), jnp.float32),
            pltpu.SemaphoreType.DMA,
            pltpu.SemaphoreType.DMA,
        ],
    )(_gather_body)
    return run(cid, initial_mean, initial_std)


def _norm_body(x_ref, m_ref, r_ref, o_ref):
    m = m_ref[...][:, :D].T[None, :, :]
    r = r_ref[...][:, :D].T[None, :, :]
    o_ref[...] = (x_ref[...] - m) * r


def _tc_normalize(xt, m, r):
    return pl.pallas_call(
        _norm_body,
        out_shape=jax.ShapeDtypeStruct((S, D, B), jnp.float32),
        grid=(B // BB,),
        in_specs=[
            pl.BlockSpec((S, D, BB), lambda i: (0, 0, i)),
            pl.BlockSpec((BB, DP), lambda i: (i, 0)),
            pl.BlockSpec((BB, DP), lambda i: (i, 0)),
        ],
        out_specs=pl.BlockSpec((S, D, BB), lambda i: (0, 0, i)),
    )(xt, m, r)


@jax.jit
def kernel(x, cluster_id, initial_mean, initial_std):
    cid = cluster_id.reshape(B)
    m, r = _sc_gather(cid, initial_mean, initial_std)
    xt = jnp.transpose(x, (1, 2, 0))
    out_t = _tc_normalize(xt, m, r)
    return jnp.transpose(out_t, (2, 0, 1))
